# baseline (speedup 1.0000x reference)
import jax
import jax.numpy as jnp
from jax import lax
from jax.experimental import pallas as pl
from jax.experimental.pallas import tpu as pltpu
from jax.experimental.pallas import tpu_sc as plsc

D_MODEL = 1024
SEQ = 2048
BATCH = 4
LANES = 16

_NC = 2
_NS = 16
_NW = _NC * _NS
_SPW = SEQ // _NW
_C = 8
_NCHUNK = _SPW // _C
_NB = 3


def _sc_body(x_hbm, pos_hbm, out_hbm,
             xb0, xb1, xb2, pb0, pb1, pb2,
             sx0, sx1, sx2, sp0, sp1, sp2, so0, so1, so2):
    wid = lax.axis_index("s") * _NC + lax.axis_index("c")
    base = wid * _SPW
    xbufs, pbufs = (xb0, xb1, xb2), (pb0, pb1, pb2)
    sxs, sps, sos = (sx0, sx1, sx2), (sp0, sp1, sp2), (so0, so1, so2)

    def issue_in(ci):
        b = ci % _NB
        s0 = base + ci * _C
        pltpu.async_copy(x_hbm.at[pl.ds(s0, _C)], xbufs[b], sxs[b])
        pltpu.async_copy(pos_hbm.at[pl.ds(s0, _C)], pbufs[b], sps[b])

    def wait_in(ci):
        b = ci % _NB
        s0 = base + ci * _C
        pltpu.make_async_copy(x_hbm.at[pl.ds(s0, _C)], xbufs[b], sxs[b]).wait()
        pltpu.make_async_copy(pos_hbm.at[pl.ds(s0, _C)], pbufs[b], sps[b]).wait()

    def issue_out(ci):
        b = ci % _NB
        s0 = base + ci * _C
        pltpu.async_copy(xbufs[b], out_hbm.at[pl.ds(s0, _C)], sos[b])

    def wait_out(ci):
        b = ci % _NB
        s0 = base + ci * _C
        pltpu.make_async_copy(xbufs[b], out_hbm.at[pl.ds(s0, _C)], sos[b]).wait()

    issue_in(0)
    issue_in(1)

    for ci in range(_NCHUNK):
        b = ci % _NB
        wait_in(ci)

        @plsc.parallel_loop(0, D_MODEL // LANES, 1, unroll=2)
        def _k(k, b=b):
            d0 = k * LANES
            for j in range(_C):
                p = pbufs[b][j, pl.ds(d0, LANES)]
                for bb in range(BATCH):
                    xbufs[b][j, bb, pl.ds(d0, LANES)] += p

        issue_out(ci)
        if ci + 2 < _NCHUNK:
            if ci >= 1:
                wait_out(ci - 1)
            issue_in(ci + 2)

    for ci in range(_NCHUNK - 3, _NCHUNK):
        wait_out(ci)


def kernel(x, pos_table):
    mesh = plsc.VectorSubcoreMesh(core_axis_name="c", subcore_axis_name="s")
    run = pl.kernel(
        _sc_body,
        mesh=mesh,
        out_type=jax.ShapeDtypeStruct((SEQ, BATCH, D_MODEL), jnp.float32),
        scratch_types=(
            [pltpu.VMEM((_C, BATCH, D_MODEL), jnp.float32)] * 3
            + [pltpu.VMEM((_C, D_MODEL), jnp.float32)] * 3
            + [pltpu.SemaphoreType.DMA] * 9
        ),
    )
    return run(x, pos_table)


# speedup vs baseline: 1.0821x; 1.0821x over previous
import jax
import jax.numpy as jnp
from jax import lax
from jax.experimental import pallas as pl
from jax.experimental.pallas import tpu as pltpu
from jax.experimental.pallas import tpu_sc as plsc

D_MODEL = 1024
SEQ = 2048
BATCH = 4
LANES = 16

_NC = 2
_NS = 16
_NW = _NC * _NS
_SPW = SEQ // _NW
_C = 4
_NCHUNK = _SPW // _C
_NB = 3


def _sc_body(x_hbm, pos_hbm, out_hbm,
             xb0, xb1, xb2, pb0, pb1, pb2, ob0, ob1, ob2,
             sx0, sx1, sx2, sp0, sp1, sp2, so0, so1, so2):
    wid = lax.axis_index("s") * _NC + lax.axis_index("c")
    base = wid * _SPW
    xbufs, pbufs, obufs = (xb0, xb1, xb2), (pb0, pb1, pb2), (ob0, ob1, ob2)
    sxs, sps, sos = (sx0, sx1, sx2), (sp0, sp1, sp2), (so0, so1, so2)

    def issue_in(ci):
        b = ci % _NB
        s0 = base + ci * _C
        pltpu.async_copy(x_hbm.at[pl.ds(s0, _C)], xbufs[b], sxs[b])
        pltpu.async_copy(pos_hbm.at[pl.ds(s0, _C)], pbufs[b], sps[b])

    def wait_in(ci):
        b = ci % _NB
        s0 = base + ci * _C
        pltpu.make_async_copy(x_hbm.at[pl.ds(s0, _C)], xbufs[b], sxs[b]).wait()
        pltpu.make_async_copy(pos_hbm.at[pl.ds(s0, _C)], pbufs[b], sps[b]).wait()

    def issue_out(ci):
        b = ci % _NB
        s0 = base + ci * _C
        pltpu.async_copy(obufs[b], out_hbm.at[pl.ds(s0, _C)], sos[b])

    def wait_out(ci):
        b = ci % _NB
        s0 = base + ci * _C
        pltpu.make_async_copy(obufs[b], out_hbm.at[pl.ds(s0, _C)], sos[b]).wait()

    issue_in(0)
    issue_in(1)

    for ci in range(_NCHUNK):
        b = ci % _NB
        wait_in(ci)
        if ci + 2 < _NCHUNK:
            issue_in(ci + 2)
        if ci >= _NB:
            wait_out(ci - _NB)

        @plsc.parallel_loop(0, D_MODEL // LANES, 1, unroll=2)
        def _k(k, b=b):
            d0 = k * LANES
            for j in range(_C):
                p = pbufs[b][j, pl.ds(d0, LANES)]
                for bb in range(BATCH):
                    obufs[b][j, bb, pl.ds(d0, LANES)] = (
                        xbufs[b][j, bb, pl.ds(d0, LANES)] + p)

        issue_out(ci)

    for ci in range(_NCHUNK - _NB, _NCHUNK):
        wait_out(ci)


def kernel(x, pos_table):
    mesh = plsc.VectorSubcoreMesh(core_axis_name="c", subcore_axis_name="s")
    run = pl.kernel(
        _sc_body,
        mesh=mesh,
        out_type=jax.ShapeDtypeStruct((SEQ, BATCH, D_MODEL), jnp.float32),
        scratch_types=(
            [pltpu.VMEM((_C, BATCH, D_MODEL), jnp.float32)] * 3
            + [pltpu.VMEM((_C, D_MODEL), jnp.float32)] * 3
            + [pltpu.VMEM((_C, BATCH, D_MODEL), jnp.float32)] * 3
            + [pltpu.SemaphoreType.DMA] * 9
        ),
    )
    return run(x, pos_table)


# speedup vs baseline: 1.2626x; 1.1668x over previous
import jax
import jax.numpy as jnp
from jax import lax
from jax.experimental import pallas as pl
from jax.experimental.pallas import tpu as pltpu
from jax.experimental.pallas import tpu_sc as plsc

D_MODEL = 1024
SEQ = 2048
BATCH = 4
LANES = 16

_NC = 2
_NS = 16
_NW = _NC * _NS
_SPW = SEQ // _NW
_C = 4
_NCHUNK = _SPW // _C
_G = _NCHUNK // 2


def _sc_body(x_hbm, pos_hbm, out_hbm,
             xb0, xb1, pb0, pb1, ob0, ob1,
             sx0, sx1, sp0, sp1, so0, so1):
    wid = lax.axis_index("s") * _NC + lax.axis_index("c")
    base = wid * _SPW
    xbufs, pbufs, obufs = (xb0, xb1), (pb0, pb1), (ob0, ob1)
    sxs, sps, sos = (sx0, sx1), (sp0, sp1), (so0, so1)

    def issue_in(ci, b):
        s0 = base + ci * _C
        pltpu.async_copy(x_hbm.at[pl.ds(s0, _C)], xbufs[b], sxs[b])
        pltpu.async_copy(pos_hbm.at[pl.ds(s0, _C)], pbufs[b], sps[b])

    issue_in(0, 0)
    issue_in(1, 1)

    def g_body(g, carry):
        for b in range(2):
            ci = g * 2 + b
            s0 = base + ci * _C
            pltpu.make_async_copy(
                x_hbm.at[pl.ds(s0, _C)], xbufs[b], sxs[b]).wait()
            pltpu.make_async_copy(
                pos_hbm.at[pl.ds(s0, _C)], pbufs[b], sps[b]).wait()

            @pl.when(g >= 1)
            def _wait_prev_out(b=b, s0=s0):
                pltpu.make_async_copy(
                    obufs[b], out_hbm.at[pl.ds(s0, _C)], sos[b]).wait()

            @plsc.parallel_loop(0, D_MODEL // LANES, 1, unroll=1)
            def _k(k, b=b):
                d0 = k * LANES
                for j in range(_C):
                    p = pbufs[b][j, pl.ds(d0, LANES)]
                    for bb in range(BATCH):
                        obufs[b][j, bb, pl.ds(d0, LANES)] = (
                            xbufs[b][j, bb, pl.ds(d0, LANES)] + p)
            pltpu.async_copy(obufs[b], out_hbm.at[pl.ds(s0, _C)], sos[b])

            @pl.when(g < _G - 1)
            def _prefetch(ci=ci, b=b):
                issue_in(ci + 2, b)
        return carry

    lax.fori_loop(0, _G, g_body, 0)
    for b in range(2):
        s0 = base + (_NCHUNK - 2 + b) * _C
        pltpu.make_async_copy(
            obufs[b], out_hbm.at[pl.ds(s0, _C)], sos[b]).wait()


def kernel(x, pos_table):
    mesh = plsc.VectorSubcoreMesh(core_axis_name="c", subcore_axis_name="s")
    run = pl.kernel(
        _sc_body,
        mesh=mesh,
        out_type=jax.ShapeDtypeStruct((SEQ, BATCH, D_MODEL), jnp.float32),
        scratch_types=[
            pltpu.VMEM((_C, BATCH, D_MODEL), jnp.float32),
            pltpu.VMEM((_C, BATCH, D_MODEL), jnp.float32),
            pltpu.VMEM((_C, D_MODEL), jnp.float32),
            pltpu.VMEM((_C, D_MODEL), jnp.float32),
            pltpu.VMEM((_C, BATCH, D_MODEL), jnp.float32),
            pltpu.VMEM((_C, BATCH, D_MODEL), jnp.float32),
            pltpu.SemaphoreType.DMA,
            pltpu.SemaphoreType.DMA,
            pltpu.SemaphoreType.DMA,
            pltpu.SemaphoreType.DMA,
            pltpu.SemaphoreType.DMA,
            pltpu.SemaphoreType.DMA,
        ],
    )
    return run(x, pos_table)
